# baseline (speedup 1.0000x reference)
import functools

import jax
import jax.numpy as jnp
from jax import lax
from jax.experimental import pallas as pl
from jax.experimental.pallas import tpu as pltpu
from jax.experimental.pallas import tpu_sc as plsc

D = 1024
H = 16
KV = 4
DH = 64
HALF = DH // 2
THETA = 10000000.0
E = 8
MOE_H = 768
BLK = 128
NB_MAX = 40
NSLOT = NB_MAX * BLK
NEG = -1e9


def _sc_gather(table, idx):
    V, d = table.shape
    B = idx.shape[0]
    info = plsc.get_sparse_core_info()
    nw = info.num_cores * info.num_subcores
    b_per_w = B // nw
    ch = min(b_per_w, 32)
    mesh = plsc.VectorSubcoreMesh(core_axis_name="c", subcore_axis_name="s")

    @functools.partial(
        pl.kernel,
        mesh=mesh,
        out_type=jax.ShapeDtypeStruct((B, d), table.dtype),
        scratch_types=[
            pltpu.VMEM((ch,), jnp.int32),
            pltpu.VMEM((ch, d), table.dtype),
            pltpu.SemaphoreType.DMA,
        ],
    )
    def k(table_hbm, idx_hbm, out_hbm, idx_v, rows_v, sem):
        wid = lax.axis_index("s") * info.num_cores + lax.axis_index("c")
        base = wid * b_per_w

        @pl.loop(0, b_per_w, step=ch)
        def _(c):
            pltpu.sync_copy(idx_hbm.at[pl.ds(base + c, ch)], idx_v)
            pltpu.async_copy(table_hbm.at[idx_v], rows_v, sem).wait()
            pltpu.sync_copy(rows_v, out_hbm.at[pl.ds(base + c, ch)])

    return k(table, idx)


def _sc_scatter_pairs(src, idx):
    T, d = src.shape
    P = idx.shape[0]
    info = plsc.get_sparse_core_info()
    nw = info.num_cores * info.num_subcores
    p_per_w = P // nw
    ch = min(p_per_w, 32)
    mesh = plsc.VectorSubcoreMesh(core_axis_name="c", subcore_axis_name="s")

    @functools.partial(
        pl.kernel,
        mesh=mesh,
        out_type=jax.ShapeDtypeStruct((NSLOT, d), src.dtype),
        scratch_types=[
            pltpu.VMEM((ch,), jnp.int32),
            pltpu.VMEM((ch, d), src.dtype),
            pltpu.SemaphoreType.DMA,
        ],
    )
    def k(src_hbm, idx_hbm, out_hbm, idx_v, rows_v, sem):
        wid = lax.axis_index("s") * info.num_cores + lax.axis_index("c")
        base = wid * p_per_w
        src_base = jnp.where(base >= T, base - T, base)

        @pl.loop(0, p_per_w, step=ch)
        def _(c):
            pltpu.sync_copy(src_hbm.at[pl.ds(src_base + c, ch)], rows_v)
            pltpu.sync_copy(idx_hbm.at[pl.ds(base + c, ch)], idx_v)
            pltpu.sync_copy(rows_v, out_hbm.at[idx_v])

    return k(src, idx)


def _bf(x):
    return x.astype(jnp.bfloat16)


def _dot3g(a, b, dims):
    return lax.dot_general(a.astype(jnp.bfloat16), b.astype(jnp.bfloat16),
                           dims, preferred_element_type=jnp.float32)


_MM = (((1,), (0,)), ((), ()))


def _doth(a, b):
    return _dot3g(a, b, _MM)


def _split(a):
    ah = a.astype(jnp.bfloat16)
    al = (a - ah.astype(jnp.float32)).astype(jnp.bfloat16)
    return ah, al


def _dot3p(ah, al, b, dims=_MM):
    bh, bl = _split(b)

    def d(x, y):
        return lax.dot_general(x, y, dims,
                               preferred_element_type=jnp.float32)

    return d(ah, bh) + d(al, bh) + d(ah, bl)


def _rms_bf16(x, g):
    var = jnp.mean(x * x, axis=-1, keepdims=True)
    return _bf(x * lax.rsqrt(var + 1e-6) * g)


def _qkv_body(x_ref, g_ref, wq_ref, wk_ref, wv_ref, cos_ref, sin_ref,
              q_ref, k_ref, v_ref):
    x = x_ref[...]
    var = jnp.mean(x * x, axis=-1, keepdims=True)
    xn = x * lax.rsqrt(var + 1e-6) * g_ref[...]
    q = _doth(xn, wq_ref[...])
    k = _doth(xn, wk_ref[...])
    v = _doth(xn, wv_ref[...])
    cos = cos_ref[...]
    sin = sin_ref[...]

    def rope(h):
        x1 = h[:, :HALF]
        x2 = h[:, HALF:]
        return jnp.concatenate(
            [x1 * cos - x2 * sin, x2 * cos + x1 * sin], axis=1)

    qh = [rope(q[:, DH * h:DH * (h + 1)]) for h in range(H)]
    q_ref[...] = jnp.concatenate(qh, axis=1)
    kh = [rope(k[:, DH * j:DH * (j + 1)]) for j in range(KV)]
    k_ref[...] = jnp.concatenate([kh[h * KV // H] for h in range(H)], axis=1)
    vh = [v[:, DH * j:DH * (j + 1)] for j in range(KV)]
    v_ref[...] = jnp.concatenate([vh[h * KV // H] for h in range(H)], axis=1)


def _qkv_call(x, g, wq, wk, wv, cos, sin):
    T = x.shape[0]
    BT = 512
    row = pl.BlockSpec((BT, D), lambda i: (i, 0))
    return pl.pallas_call(
        _qkv_body,
        grid=(T // BT,),
        in_specs=[
            row,
            pl.BlockSpec((1, D), lambda i: (0, 0)),
            pl.BlockSpec((D, H * DH), lambda i: (0, 0)),
            pl.BlockSpec((D, KV * DH), lambda i: (0, 0)),
            pl.BlockSpec((D, KV * DH), lambda i: (0, 0)),
            pl.BlockSpec((BT, HALF), lambda i: (i, 0)),
            pl.BlockSpec((BT, HALF), lambda i: (i, 0)),
        ],
        out_specs=[row, row, row],
        out_shape=[
            jax.ShapeDtypeStruct((T, H * DH), jnp.float32),
            jax.ShapeDtypeStruct((T, H * DH), jnp.float32),
            jax.ShapeDtypeStruct((T, H * DH), jnp.float32),
        ],
    )(x, g, wq, wk, wv, cos, sin)


def _attn_body(q_ref, k_ref, v_ref, o_ref, *, T, BQ):
    i = pl.program_id(1)
    qbase = i * BQ
    rows = qbase + lax.broadcasted_iota(jnp.int32, (BQ, T), 0)
    cols = lax.broadcasted_iota(jnp.int32, (BQ, T), 1)
    mask = cols <= rows
    outs = []
    for h in range(2):
        q = q_ref[:, DH * h:DH * (h + 1)]
        k = k_ref[:, DH * h:DH * (h + 1)]
        s = _dot3g(q, k, (((1,), (1,)), ((), ())))
        s = s * (1.0 / 8.0)
        s = jnp.where(mask, s, NEG)
        m = jnp.max(s, axis=1, keepdims=True)
        p = jnp.exp(s - m)
        l = jnp.sum(p, axis=1, keepdims=True)
        v = v_ref[:, DH * h:DH * (h + 1)]
        o = _dot3g(p, v, (((1,), (0,)), ((), ())))
        outs.append(o / l)
    o_ref[...] = jnp.concatenate(outs, axis=1)


def _attn_call(q, k, v):
    T = q.shape[0]
    BQ = 256
    return pl.pallas_call(
        functools.partial(_attn_body, T=T, BQ=BQ),
        grid=(H // 2, T // BQ),
        in_specs=[
            pl.BlockSpec((BQ, 2 * DH), lambda g, i: (i, g)),
            pl.BlockSpec((T, 2 * DH), lambda g, i: (0, g)),
            pl.BlockSpec((T, 2 * DH), lambda g, i: (0, g)),
        ],
        out_specs=pl.BlockSpec((BQ, 2 * DH), lambda g, i: (i, g)),
        out_shape=jax.ShapeDtypeStruct((T, H * DH), jnp.float32),
    )(q, k, v)


def _wo_body(a_ref, w_ref, res_ref, o_ref):
    o_ref[...] = res_ref[...] + _doth(a_ref[...], w_ref[...])


def _wo_call(a, w, res):
    T = a.shape[0]
    BT = 512
    return pl.pallas_call(
        _wo_body,
        grid=(T // BT,),
        in_specs=[
            pl.BlockSpec((BT, H * DH), lambda i: (i, 0)),
            pl.BlockSpec((H * DH, D), lambda i: (0, 0)),
            pl.BlockSpec((BT, D), lambda i: (i, 0)),
        ],
        out_specs=pl.BlockSpec((BT, D), lambda i: (i, 0)),
        out_shape=jax.ShapeDtypeStruct((T, D), jnp.float32),
    )(a, w, res)


def _ffn_body(x_ref, g_ref, wg_ref, wu_ref, wd_ref, o_ref, xh_ref, xl_ref):
    j = pl.program_id(0)

    @pl.when(j == 0)
    def _():
        x = x_ref[...]
        var = jnp.mean(x * x, axis=-1, keepdims=True)
        xh, xl = _split(x * lax.rsqrt(var + 1e-6) * g_ref[...])
        xh_ref[...] = xh
        xl_ref[...] = xl
        o_ref[...] = x

    xh = xh_ref[...]
    xl = xl_ref[...]
    gg = _dot3p(xh, xl, wg_ref[...])
    uu = _dot3p(xh, xl, wu_ref[...])
    a = jax.nn.silu(gg) * uu
    ah, al = _split(a)
    o_ref[...] += _dot3p(ah, al, wd_ref[...])


def _ffn_call(x, g, wg, wu, wd):
    T = x.shape[0]
    F = wg.shape[1]
    BF = 128
    return pl.pallas_call(
        _ffn_body,
        grid=(F // BF,),
        in_specs=[
            pl.BlockSpec((T, D), lambda j: (0, 0)),
            pl.BlockSpec((1, D), lambda j: (0, 0)),
            pl.BlockSpec((D, BF), lambda j: (0, j)),
            pl.BlockSpec((D, BF), lambda j: (0, j)),
            pl.BlockSpec((BF, D), lambda j: (j, 0)),
        ],
        out_specs=pl.BlockSpec((T, D), lambda j: (0, 0)),
        out_shape=jax.ShapeDtypeStruct((T, D), jnp.float32),
        scratch_shapes=[
            pltpu.VMEM((T, D), jnp.bfloat16),
            pltpu.VMEM((T, D), jnp.bfloat16),
        ],
    )(x, g, wg, wu, wd)


def _sublane_cumsum(c, n):
    sh = 1
    while sh < n:
        c = c + jnp.concatenate(
            [jnp.zeros((sh, c.shape[1]), c.dtype), c[:-sh, :]], axis=0)
        sh *= 2
    return c


def _route_body(x_ref, g_ref, wr_ref, h2_ref, posb_ref, wb_ref, bexp_ref):
    x = x_ref[...]
    var = jnp.mean(x * x, axis=-1, keepdims=True)
    h2 = x * lax.rsqrt(var + 1e-6) * g_ref[...]
    h2_ref[...] = h2
    T = x.shape[0]
    logits = _doth(h2, wr_ref[...])[:, :E]
    mx = jnp.max(logits, axis=1, keepdims=True)
    ex = jnp.exp(logits - mx)
    probs = ex / jnp.sum(ex, axis=1, keepdims=True)
    ii = lax.broadcasted_iota(jnp.int32, (T, E), 1)
    m1 = jnp.max(probs, axis=1, keepdims=True)
    i1 = jnp.min(jnp.where(probs == m1, ii, E), axis=1, keepdims=True)
    pm = jnp.where(ii == i1, NEG, probs)
    m2 = jnp.max(pm, axis=1, keepdims=True)
    i2 = jnp.min(jnp.where(pm == m2, ii, E), axis=1, keepdims=True)
    tot = m1 + m2
    w1 = m1 / tot
    w2 = m2 / tot
    oh0 = (ii == i1).astype(jnp.float32)
    oh1 = (ii == i2).astype(jnp.float32)
    inc0 = _sublane_cumsum(oh0, T)
    inc1 = _sublane_cumsum(oh1, T)
    exc0 = inc0 - oh0
    exc1 = inc1 - oh1
    tot0 = inc0[T - 1:T, :]
    tot1 = inc1[T - 1:T, :]
    count = tot0 + tot1
    nbpad = jnp.ceil(count / BLK) * BLK
    incb = nbpad
    sh = 1
    while sh < E:
        incb = incb + jnp.concatenate(
            [jnp.zeros((1, sh), jnp.float32), incb[:, :-sh]], axis=1)
        sh *= 2
    off = incb - nbpad
    pos0 = jnp.sum(oh0 * (off + exc0), axis=1, keepdims=True)
    pos1 = jnp.sum(oh1 * (off + tot0 + exc1), axis=1, keepdims=True)
    lane0 = (ii == 0).astype(jnp.float32)
    lane1 = (ii == 1).astype(jnp.float32)
    posb_ref[...] = (pos0 * lane0 + pos1 * lane1).astype(jnp.int32)
    wb_ref[...] = w1 * lane0 + w2 * lane1
    seg_end = incb
    brow = lax.broadcasted_iota(
        jnp.int32, (NB_MAX, E), 0).astype(jnp.float32) * BLK
    mb = (seg_end <= brow).astype(jnp.float32)
    bexp = jnp.minimum(jnp.sum(mb, axis=1, keepdims=True), E - 1)
    blane0 = (lax.broadcasted_iota(jnp.int32, (NB_MAX, E), 1) == 0)
    bexp_ref[...] = (bexp * blane0.astype(jnp.float32)).astype(jnp.int32)


def _route_call(x, g, wr_pad):
    T = x.shape[0]
    return pl.pallas_call(
        _route_body,
        in_specs=[
            pl.BlockSpec((T, D), lambda: (0, 0)),
            pl.BlockSpec((1, D), lambda: (0, 0)),
            pl.BlockSpec((D, 128), lambda: (0, 0)),
        ],
        out_specs=[
            pl.BlockSpec((T, D), lambda: (0, 0)),
            pl.BlockSpec((T, E), lambda: (0, 0)),
            pl.BlockSpec((T, E), lambda: (0, 0)),
            pl.BlockSpec((NB_MAX, E), lambda: (0, 0)),
        ],
        out_shape=[
            jax.ShapeDtypeStruct((T, D), jnp.float32),
            jax.ShapeDtypeStruct((T, E), jnp.int32),
            jax.ShapeDtypeStruct((T, E), jnp.float32),
            jax.ShapeDtypeStruct((NB_MAX, E), jnp.int32),
        ],
    )(x, g, wr_pad)


def _moe_body(be_ref, x_ref, w1_ref, w2_ref, o_ref):
    xb = _bf(x_ref[...])
    h = jnp.dot(xb, _bf(w1_ref[0]), preferred_element_type=jnp.float32)
    h = _bf(jax.nn.silu(h))
    o_ref[...] = jnp.dot(h, _bf(w2_ref[0]), preferred_element_type=jnp.float32)


def _moe_call(bexp, xsorted, w1, w2):
    grid_spec = pltpu.PrefetchScalarGridSpec(
        num_scalar_prefetch=1,
        grid=(NB_MAX,),
        in_specs=[
            pl.BlockSpec((BLK, D), lambda b, be: (b, 0)),
            pl.BlockSpec((1, D, MOE_H), lambda b, be: (be[b], 0, 0)),
            pl.BlockSpec((1, MOE_H, D), lambda b, be: (be[b], 0, 0)),
        ],
        out_specs=pl.BlockSpec((BLK, D), lambda b, be: (b, 0)),
    )
    return pl.pallas_call(
        _moe_body,
        grid_spec=grid_spec,
        out_shape=jax.ShapeDtypeStruct((NSLOT, D), jnp.float32),
    )(bexp, xsorted, w1, w2)


def _combine_body(wb_ref, x_ref, y0_ref, y1_ref, o_ref):
    w0 = _bf(wb_ref[:, 0:1]).astype(jnp.float32)
    w1 = _bf(wb_ref[:, 1:2]).astype(jnp.float32)
    y0 = _bf(y0_ref[...]).astype(jnp.float32)
    y1 = _bf(y1_ref[...]).astype(jnp.float32)
    o_ref[...] = x_ref[...] + w0 * y0 + w1 * y1


def _combine_call(wb, x, ycomb):
    T = x.shape[0]
    BT = 512
    nb = T // BT
    return pl.pallas_call(
        _combine_body,
        grid=(nb,),
        in_specs=[
            pl.BlockSpec((BT, E), lambda i: (i, 0)),
            pl.BlockSpec((BT, D), lambda i: (i, 0)),
            pl.BlockSpec((BT, D), lambda i: (i, 0)),
            pl.BlockSpec((BT, D), lambda i: (i + nb, 0)),
        ],
        out_specs=pl.BlockSpec((BT, D), lambda i: (i, 0)),
        out_shape=jax.ShapeDtypeStruct((T, D), jnp.float32),
    )(wb, x, ycomb, ycomb)


def _lmhead_body(x_ref, g_ref, emb_ref, o_ref, xn_ref):
    j = pl.program_id(0)

    @pl.when(j == 0)
    def _():
        xn_ref[...] = _rms_bf16(x_ref[...], g_ref[...])

    e = _bf(emb_ref[...])
    o_ref[...] = lax.dot_general(xn_ref[...], e, (((1,), (1,)), ((), ())),
                                 preferred_element_type=jnp.float32)


def _lmhead_call(x, g, emb):
    T = x.shape[0]
    V = emb.shape[0]
    BN = 512
    return pl.pallas_call(
        _lmhead_body,
        grid=(V // BN,),
        in_specs=[
            pl.BlockSpec((T, D), lambda j: (0, 0)),
            pl.BlockSpec((1, D), lambda j: (0, 0)),
            pl.BlockSpec((BN, D), lambda j: (j, 0)),
        ],
        out_specs=pl.BlockSpec((T, BN), lambda j: (0, j)),
        out_shape=jax.ShapeDtypeStruct((T, V), jnp.float32),
        scratch_shapes=[pltpu.VMEM((T, D), jnp.bfloat16)],
    )(x, g, emb)


def kernel(token_ids, params):
    emb = params["emb"]
    b, s = token_ids.shape
    T = b * s
    ids = token_ids.reshape(T).astype(jnp.int32)

    x = _sc_gather(emb, ids)

    inv_freq = 1.0 / (THETA ** (jnp.arange(HALF, dtype=jnp.float32) / HALF))
    ang = jnp.arange(s, dtype=jnp.float32)[:, None] * inv_freq[None, :]
    cos = jnp.tile(jnp.cos(ang), (b, 1))
    sin = jnp.tile(jnp.sin(ang), (b, 1))

    for p in params["layers"]:
        q, k, v = _qkv_call(x, p["ln1"].reshape(1, D), p["Wq"], p["Wk"],
                            p["Wv"], cos, sin)
        attn = _attn_call(q, k, v)
        x = _wo_call(attn, p["Wo"], x)
        if "Wr" in p:
            wr_pad = jnp.pad(p["Wr"], ((0, 0), (0, 128 - E)))
            h2, posb, wb, bexpb = _route_call(x, p["ln2"].reshape(1, D),
                                              wr_pad)
            pos_flat = jnp.concatenate([posb[:, 0], posb[:, 1]], axis=0)
            bexp = bexpb[:, 0]
            xsorted = _sc_scatter_pairs(h2, pos_flat)
            ysorted = _moe_call(bexp, xsorted, p["W1"], p["W2"])
            ycomb = _sc_gather(ysorted, pos_flat)
            x = _combine_call(wb, x, ycomb)
        else:
            x = _ffn_call(x, p["ln2"].reshape(1, D), p["Wg"], p["Wu"],
                          p["Wd"])

    out = _lmhead_call(x, params["final"].reshape(1, D), emb)
    return out.reshape(b, s, emb.shape[0])


# speedup vs baseline: 1.0144x; 1.0144x over previous
import functools

import jax
import jax.numpy as jnp
from jax import lax
from jax.experimental import pallas as pl
from jax.experimental.pallas import tpu as pltpu
from jax.experimental.pallas import tpu_sc as plsc

D = 1024
H = 16
KV = 4
DH = 64
HALF = DH // 2
THETA = 10000000.0
E = 8
MOE_H = 768
BLK = 128
NB_MAX = 40
NSLOT = NB_MAX * BLK
NEG = -1e9


def _sc_gather(table, idx):
    V, d = table.shape
    B = idx.shape[0]
    info = plsc.get_sparse_core_info()
    nw = info.num_cores * info.num_subcores
    b_per_w = B // nw
    ch = min(b_per_w, 64)
    mesh = plsc.VectorSubcoreMesh(core_axis_name="c", subcore_axis_name="s")

    @functools.partial(
        pl.kernel,
        mesh=mesh,
        out_type=jax.ShapeDtypeStruct((B, d), table.dtype),
        scratch_types=[
            pltpu.VMEM((ch,), jnp.int32),
            pltpu.VMEM((ch, d), table.dtype),
            pltpu.SemaphoreType.DMA,
        ],
    )
    def k(table_hbm, idx_hbm, out_hbm, idx_v, rows_v, sem):
        wid = lax.axis_index("s") * info.num_cores + lax.axis_index("c")
        base = wid * b_per_w

        @pl.loop(0, b_per_w, step=ch)
        def _(c):
            pltpu.sync_copy(idx_hbm.at[pl.ds(base + c, ch)], idx_v)
            pltpu.async_copy(table_hbm.at[idx_v], rows_v, sem).wait()
            pltpu.sync_copy(rows_v, out_hbm.at[pl.ds(base + c, ch)])

    return k(table, idx)


def _sc_scatter_pairs(src, idx):
    T, d = src.shape
    P = idx.shape[0]
    info = plsc.get_sparse_core_info()
    nw = info.num_cores * info.num_subcores
    p_per_w = P // nw
    ch = min(p_per_w, 64)
    mesh = plsc.VectorSubcoreMesh(core_axis_name="c", subcore_axis_name="s")

    @functools.partial(
        pl.kernel,
        mesh=mesh,
        out_type=jax.ShapeDtypeStruct((NSLOT, d), src.dtype),
        scratch_types=[
            pltpu.VMEM((ch,), jnp.int32),
            pltpu.VMEM((ch, d), src.dtype),
            pltpu.SemaphoreType.DMA,
        ],
    )
    def k(src_hbm, idx_hbm, out_hbm, idx_v, rows_v, sem):
        wid = lax.axis_index("s") * info.num_cores + lax.axis_index("c")
        base = wid * p_per_w
        src_base = jnp.where(base >= T, base - T, base)

        @pl.loop(0, p_per_w, step=ch)
        def _(c):
            pltpu.sync_copy(src_hbm.at[pl.ds(src_base + c, ch)], rows_v)
            pltpu.sync_copy(idx_hbm.at[pl.ds(base + c, ch)], idx_v)
            pltpu.sync_copy(rows_v, out_hbm.at[idx_v])

    return k(src, idx)


def _bf(x):
    return x.astype(jnp.bfloat16)


def _dot3g(a, b, dims):
    return lax.dot_general(a.astype(jnp.bfloat16), b.astype(jnp.bfloat16),
                           dims, preferred_element_type=jnp.float32)


_MM = (((1,), (0,)), ((), ()))


def _doth(a, b):
    return _dot3g(a, b, _MM)


def _split(a):
    ah = a.astype(jnp.bfloat16)
    al = (a - ah.astype(jnp.float32)).astype(jnp.bfloat16)
    return ah, al


def _dot3p(ah, al, b, dims=_MM):
    bh, bl = _split(b)

    def d(x, y):
        return lax.dot_general(x, y, dims,
                               preferred_element_type=jnp.float32)

    return d(ah, bh) + d(al, bh) + d(ah, bl)


def _rms_bf16(x, g):
    var = jnp.mean(x * x, axis=-1, keepdims=True)
    return _bf(x * lax.rsqrt(var + 1e-6) * g)


def _qkv_body(x_ref, g_ref, wq_ref, wk_ref, wv_ref, cos_ref, sin_ref,
              q_ref, k_ref, v_ref):
    x = x_ref[...]
    var = jnp.mean(x * x, axis=-1, keepdims=True)
    xn = x * lax.rsqrt(var + 1e-6) * g_ref[...]
    q = _doth(xn, wq_ref[...])
    k = _doth(xn, wk_ref[...])
    v = _doth(xn, wv_ref[...])
    cos = cos_ref[...]
    sin = sin_ref[...]

    def rope(h):
        x1 = h[:, :HALF]
        x2 = h[:, HALF:]
        return jnp.concatenate(
            [x1 * cos - x2 * sin, x2 * cos + x1 * sin], axis=1)

    qh = [rope(q[:, DH * h:DH * (h + 1)]) for h in range(H)]
    q_ref[...] = jnp.concatenate(qh, axis=1)
    kh = [rope(k[:, DH * j:DH * (j + 1)]) for j in range(KV)]
    k_ref[...] = jnp.concatenate([kh[h * KV // H] for h in range(H)], axis=1)
    vh = [v[:, DH * j:DH * (j + 1)] for j in range(KV)]
    v_ref[...] = jnp.concatenate([vh[h * KV // H] for h in range(H)], axis=1)


def _qkv_call(x, g, wq, wk, wv, cos, sin):
    T = x.shape[0]
    BT = 512
    row = pl.BlockSpec((BT, D), lambda i: (i, 0))
    return pl.pallas_call(
        _qkv_body,
        grid=(T // BT,),
        in_specs=[
            row,
            pl.BlockSpec((1, D), lambda i: (0, 0)),
            pl.BlockSpec((D, H * DH), lambda i: (0, 0)),
            pl.BlockSpec((D, KV * DH), lambda i: (0, 0)),
            pl.BlockSpec((D, KV * DH), lambda i: (0, 0)),
            pl.BlockSpec((BT, HALF), lambda i: (i, 0)),
            pl.BlockSpec((BT, HALF), lambda i: (i, 0)),
        ],
        out_specs=[row, row, row],
        out_shape=[
            jax.ShapeDtypeStruct((T, H * DH), jnp.float32),
            jax.ShapeDtypeStruct((T, H * DH), jnp.float32),
            jax.ShapeDtypeStruct((T, H * DH), jnp.float32),
        ],
    )(x, g, wq, wk, wv, cos, sin)


def _attn_body(q_ref, k_ref, v_ref, o_ref, *, T, BQ):
    i = pl.program_id(1)
    qbase = i * BQ
    rows = qbase + lax.broadcasted_iota(jnp.int32, (BQ, T), 0)
    cols = lax.broadcasted_iota(jnp.int32, (BQ, T), 1)
    mask = cols <= rows
    outs = []
    for h in range(2):
        q = q_ref[:, DH * h:DH * (h + 1)]
        k = k_ref[:, DH * h:DH * (h + 1)]
        s = _dot3g(q, k, (((1,), (1,)), ((), ())))
        s = s * (1.0 / 8.0)
        s = jnp.where(mask, s, NEG)
        m = jnp.max(s, axis=1, keepdims=True)
        p = jnp.exp(s - m)
        l = jnp.sum(p, axis=1, keepdims=True)
        v = v_ref[:, DH * h:DH * (h + 1)]
        o = _dot3g(p, v, (((1,), (0,)), ((), ())))
        outs.append(o / l)
    o_ref[...] = jnp.concatenate(outs, axis=1)


def _attn_call(q, k, v):
    T = q.shape[0]
    BQ = 256
    return pl.pallas_call(
        functools.partial(_attn_body, T=T, BQ=BQ),
        grid=(H // 2, T // BQ),
        in_specs=[
            pl.BlockSpec((BQ, 2 * DH), lambda g, i: (i, g)),
            pl.BlockSpec((T, 2 * DH), lambda g, i: (0, g)),
            pl.BlockSpec((T, 2 * DH), lambda g, i: (0, g)),
        ],
        out_specs=pl.BlockSpec((BQ, 2 * DH), lambda g, i: (i, g)),
        out_shape=jax.ShapeDtypeStruct((T, H * DH), jnp.float32),
    )(q, k, v)


def _wo_body(a_ref, w_ref, res_ref, o_ref):
    o_ref[...] = res_ref[...] + _doth(a_ref[...], w_ref[...])


def _wo_call(a, w, res):
    T = a.shape[0]
    BT = 512
    return pl.pallas_call(
        _wo_body,
        grid=(T // BT,),
        in_specs=[
            pl.BlockSpec((BT, H * DH), lambda i: (i, 0)),
            pl.BlockSpec((H * DH, D), lambda i: (0, 0)),
            pl.BlockSpec((BT, D), lambda i: (i, 0)),
        ],
        out_specs=pl.BlockSpec((BT, D), lambda i: (i, 0)),
        out_shape=jax.ShapeDtypeStruct((T, D), jnp.float32),
    )(a, w, res)


def _ffn_body(x_ref, g_ref, wg_ref, wu_ref, wd_ref, o_ref, xh_ref, xl_ref):
    j = pl.program_id(0)

    @pl.when(j == 0)
    def _():
        x = x_ref[...]
        var = jnp.mean(x * x, axis=-1, keepdims=True)
        xh, xl = _split(x * lax.rsqrt(var + 1e-6) * g_ref[...])
        xh_ref[...] = xh
        xl_ref[...] = xl
        o_ref[...] = x

    xh = xh_ref[...]
    xl = xl_ref[...]
    gg = _dot3p(xh, xl, wg_ref[...])
    uu = _dot3p(xh, xl, wu_ref[...])
    a = jax.nn.silu(gg) * uu
    ah, al = _split(a)
    o_ref[...] += _dot3p(ah, al, wd_ref[...])


def _ffn_call(x, g, wg, wu, wd):
    T = x.shape[0]
    F = wg.shape[1]
    BF = 128
    return pl.pallas_call(
        _ffn_body,
        grid=(F // BF,),
        in_specs=[
            pl.BlockSpec((T, D), lambda j: (0, 0)),
            pl.BlockSpec((1, D), lambda j: (0, 0)),
            pl.BlockSpec((D, BF), lambda j: (0, j)),
            pl.BlockSpec((D, BF), lambda j: (0, j)),
            pl.BlockSpec((BF, D), lambda j: (j, 0)),
        ],
        out_specs=pl.BlockSpec((T, D), lambda j: (0, 0)),
        out_shape=jax.ShapeDtypeStruct((T, D), jnp.float32),
        scratch_shapes=[
            pltpu.VMEM((T, D), jnp.bfloat16),
            pltpu.VMEM((T, D), jnp.bfloat16),
        ],
    )(x, g, wg, wu, wd)


def _sublane_cumsum(c, n):
    sh = 1
    while sh < n:
        c = c + jnp.concatenate(
            [jnp.zeros((sh, c.shape[1]), c.dtype), c[:-sh, :]], axis=0)
        sh *= 2
    return c


def _route_body(x_ref, g_ref, wr_ref, h2_ref, posb_ref, wb_ref, bexp_ref):
    x = x_ref[...]
    var = jnp.mean(x * x, axis=-1, keepdims=True)
    h2 = x * lax.rsqrt(var + 1e-6) * g_ref[...]
    h2_ref[...] = h2
    T = x.shape[0]
    logits = _doth(h2, wr_ref[...])[:, :E]
    mx = jnp.max(logits, axis=1, keepdims=True)
    ex = jnp.exp(logits - mx)
    probs = ex / jnp.sum(ex, axis=1, keepdims=True)
    ii = lax.broadcasted_iota(jnp.int32, (T, E), 1)
    m1 = jnp.max(probs, axis=1, keepdims=True)
    i1 = jnp.min(jnp.where(probs == m1, ii, E), axis=1, keepdims=True)
    pm = jnp.where(ii == i1, NEG, probs)
    m2 = jnp.max(pm, axis=1, keepdims=True)
    i2 = jnp.min(jnp.where(pm == m2, ii, E), axis=1, keepdims=True)
    tot = m1 + m2
    w1 = m1 / tot
    w2 = m2 / tot
    oh0 = (ii == i1).astype(jnp.float32)
    oh1 = (ii == i2).astype(jnp.float32)
    inc0 = _sublane_cumsum(oh0, T)
    inc1 = _sublane_cumsum(oh1, T)
    exc0 = inc0 - oh0
    exc1 = inc1 - oh1
    tot0 = inc0[T - 1:T, :]
    tot1 = inc1[T - 1:T, :]
    count = tot0 + tot1
    nbpad = jnp.ceil(count / BLK) * BLK
    incb = nbpad
    sh = 1
    while sh < E:
        incb = incb + jnp.concatenate(
            [jnp.zeros((1, sh), jnp.float32), incb[:, :-sh]], axis=1)
        sh *= 2
    off = incb - nbpad
    pos0 = jnp.sum(oh0 * (off + exc0), axis=1, keepdims=True)
    pos1 = jnp.sum(oh1 * (off + tot0 + exc1), axis=1, keepdims=True)
    lane0 = (ii == 0).astype(jnp.float32)
    lane1 = (ii == 1).astype(jnp.float32)
    posb_ref[...] = (pos0 * lane0 + pos1 * lane1).astype(jnp.int32)
    wb_ref[...] = w1 * lane0 + w2 * lane1
    seg_end = incb
    brow = lax.broadcasted_iota(
        jnp.int32, (NB_MAX, E), 0).astype(jnp.float32) * BLK
    mb = (seg_end <= brow).astype(jnp.float32)
    bexp = jnp.minimum(jnp.sum(mb, axis=1, keepdims=True), E - 1)
    blane0 = (lax.broadcasted_iota(jnp.int32, (NB_MAX, E), 1) == 0)
    bexp_ref[...] = (bexp * blane0.astype(jnp.float32)).astype(jnp.int32)


def _route_call(x, g, wr_pad):
    T = x.shape[0]
    return pl.pallas_call(
        _route_body,
        in_specs=[
            pl.BlockSpec((T, D), lambda: (0, 0)),
            pl.BlockSpec((1, D), lambda: (0, 0)),
            pl.BlockSpec((D, 128), lambda: (0, 0)),
        ],
        out_specs=[
            pl.BlockSpec((T, D), lambda: (0, 0)),
            pl.BlockSpec((T, E), lambda: (0, 0)),
            pl.BlockSpec((T, E), lambda: (0, 0)),
            pl.BlockSpec((NB_MAX, E), lambda: (0, 0)),
        ],
        out_shape=[
            jax.ShapeDtypeStruct((T, D), jnp.float32),
            jax.ShapeDtypeStruct((T, E), jnp.int32),
            jax.ShapeDtypeStruct((T, E), jnp.float32),
            jax.ShapeDtypeStruct((NB_MAX, E), jnp.int32),
        ],
    )(x, g, wr_pad)


def _moe_body(be_ref, x_ref, w1_ref, w2_ref, o_ref):
    xb = _bf(x_ref[...])
    h = jnp.dot(xb, _bf(w1_ref[0]), preferred_element_type=jnp.float32)
    h = _bf(jax.nn.silu(h))
    o_ref[...] = jnp.dot(h, _bf(w2_ref[0]), preferred_element_type=jnp.float32)


def _moe_call(bexp, xsorted, w1, w2):
    grid_spec = pltpu.PrefetchScalarGridSpec(
        num_scalar_prefetch=1,
        grid=(NB_MAX,),
        in_specs=[
            pl.BlockSpec((BLK, D), lambda b, be: (b, 0)),
            pl.BlockSpec((1, D, MOE_H), lambda b, be: (be[b], 0, 0)),
            pl.BlockSpec((1, MOE_H, D), lambda b, be: (be[b], 0, 0)),
        ],
        out_specs=pl.BlockSpec((BLK, D), lambda b, be: (b, 0)),
    )
    return pl.pallas_call(
        _moe_body,
        grid_spec=grid_spec,
        out_shape=jax.ShapeDtypeStruct((NSLOT, D), jnp.float32),
    )(bexp, xsorted, w1, w2)


def _combine_body(wb_ref, x_ref, y0_ref, y1_ref, o_ref):
    w0 = _bf(wb_ref[:, 0:1]).astype(jnp.float32)
    w1 = _bf(wb_ref[:, 1:2]).astype(jnp.float32)
    y0 = _bf(y0_ref[...]).astype(jnp.float32)
    y1 = _bf(y1_ref[...]).astype(jnp.float32)
    o_ref[...] = x_ref[...] + w0 * y0 + w1 * y1


def _combine_call(wb, x, ycomb):
    T = x.shape[0]
    BT = 512
    nb = T // BT
    return pl.pallas_call(
        _combine_body,
        grid=(nb,),
        in_specs=[
            pl.BlockSpec((BT, E), lambda i: (i, 0)),
            pl.BlockSpec((BT, D), lambda i: (i, 0)),
            pl.BlockSpec((BT, D), lambda i: (i, 0)),
            pl.BlockSpec((BT, D), lambda i: (i + nb, 0)),
        ],
        out_specs=pl.BlockSpec((BT, D), lambda i: (i, 0)),
        out_shape=jax.ShapeDtypeStruct((T, D), jnp.float32),
    )(wb, x, ycomb, ycomb)


def _lmhead_body(wb_ref, x_ref, y0_ref, y1_ref, g_ref, emb_ref, o_ref,
                 xn_ref):
    j = pl.program_id(0)

    @pl.when(j == 0)
    def _():
        w0 = _bf(wb_ref[:, 0:1]).astype(jnp.float32)
        w1 = _bf(wb_ref[:, 1:2]).astype(jnp.float32)
        y0 = _bf(y0_ref[...]).astype(jnp.float32)
        y1 = _bf(y1_ref[...]).astype(jnp.float32)
        xc = x_ref[...] + w0 * y0 + w1 * y1
        xn_ref[...] = _rms_bf16(xc, g_ref[...])

    e = _bf(emb_ref[...])
    o_ref[...] = lax.dot_general(xn_ref[...], e, (((1,), (1,)), ((), ())),
                                 preferred_element_type=jnp.float32)


def _lmhead_call(wb, x, ycomb, g, emb):
    T = x.shape[0]
    V = emb.shape[0]
    BN = 512
    return pl.pallas_call(
        _lmhead_body,
        grid=(V // BN,),
        in_specs=[
            pl.BlockSpec((T, E), lambda j: (0, 0)),
            pl.BlockSpec((T, D), lambda j: (0, 0)),
            pl.BlockSpec((T, D), lambda j: (0, 0)),
            pl.BlockSpec((T, D), lambda j: (1, 0)),
            pl.BlockSpec((1, D), lambda j: (0, 0)),
            pl.BlockSpec((BN, D), lambda j: (j, 0)),
        ],
        out_specs=pl.BlockSpec((T, BN), lambda j: (0, j)),
        out_shape=jax.ShapeDtypeStruct((T, V), jnp.float32),
        scratch_shapes=[pltpu.VMEM((T, D), jnp.bfloat16)],
    )(wb, x, ycomb, ycomb, g, emb)


def kernel(token_ids, params):
    emb = params["emb"]
    b, s = token_ids.shape
    T = b * s
    ids = token_ids.reshape(T).astype(jnp.int32)

    x = _sc_gather(emb, ids)
    moe_tail = None

    inv_freq = 1.0 / (THETA ** (jnp.arange(HALF, dtype=jnp.float32) / HALF))
    ang = jnp.arange(s, dtype=jnp.float32)[:, None] * inv_freq[None, :]
    cos = jnp.tile(jnp.cos(ang), (b, 1))
    sin = jnp.tile(jnp.sin(ang), (b, 1))

    for p in params["layers"]:
        q, k, v = _qkv_call(x, p["ln1"].reshape(1, D), p["Wq"], p["Wk"],
                            p["Wv"], cos, sin)
        attn = _attn_call(q, k, v)
        x = _wo_call(attn, p["Wo"], x)
        if "Wr" in p:
            wr_pad = jnp.pad(p["Wr"], ((0, 0), (0, 128 - E)))
            h2, posb, wb, bexpb = _route_call(x, p["ln2"].reshape(1, D),
                                              wr_pad)
            pos_flat = jnp.concatenate([posb[:, 0], posb[:, 1]], axis=0)
            bexp = bexpb[:, 0]
            xsorted = _sc_scatter_pairs(h2, pos_flat)
            ysorted = _moe_call(bexp, xsorted, p["W1"], p["W2"])
            ycomb = _sc_gather(ysorted, pos_flat)
            if p is params["layers"][-1]:
                moe_tail = (wb, ycomb)
            else:
                x = _combine_call(wb, x, ycomb)
        else:
            x = _ffn_call(x, p["ln2"].reshape(1, D), p["Wg"], p["Wu"],
                          p["Wd"])

    if moe_tail is not None:
        wb, ycomb = moe_tail
        out = _lmhead_call(wb, x, ycomb, params["final"].reshape(1, D), emb)
    else:
        zw = jnp.zeros((T, E), jnp.float32)
        zy = jnp.zeros((2 * T, D), jnp.float32)
        out = _lmhead_call(zw, x, zy, params["final"].reshape(1, D), emb)
    return out.reshape(b, s, emb.shape[0])
